# baseline (device time: 105120 ns/iter reference)
import jax
import jax.numpy as jnp
from jax import lax
from jax.experimental import pallas as pl
from jax.experimental.pallas import tpu as pltpu

N = 4
EL = 4
CAPE = 160
D = 1024
F = 2048
T = 2048
B = N * CAPE
S = EL * B
TC = 256


def _body(x_bf, slots, slots_t, w1_hbm, w2_hbm, out,
          Pj, Pc, x_send, recv_x, result_comp, result_recv,
          w1_buf, w2_buf, h, dsend, drecv, rsend, rrecv, wsem):
    me = lax.axis_index("i")

    def base(j, k):
        return (j * N + k) * CAPE

    def dispatch_rdma(j, k):
        return pltpu.make_async_remote_copy(
            src_ref=x_send.at[pl.ds(base(j, k), CAPE), :],
            dst_ref=recv_x.at[j, k],
            send_sem=dsend.at[j, k],
            recv_sem=drecv.at[j, k],
            device_id=((me + k) % N,),
            device_id_type=pl.DeviceIdType.MESH,
        )

    def return_rdma(j, k):
        return pltpu.make_async_remote_copy(
            src_ref=result_comp.at[j, k - 1],
            dst_ref=result_recv.at[pl.ds(base(j, k), CAPE), :],
            send_sem=rsend.at[j, k],
            recv_sem=rrecv.at[j, k],
            device_id=((me - k) % N,),
            device_id_type=pl.DeviceIdType.MESH,
        )

    def load_w(j):
        pltpu.make_async_copy(w1_hbm.at[j], w1_buf, wsem.at[0]).start()
        pltpu.make_async_copy(w2_hbm.at[j], w2_buf, wsem.at[1]).start()

    def wait_w(j):
        pltpu.make_async_copy(w1_hbm.at[j], w1_buf, wsem.at[0]).wait()
        pltpu.make_async_copy(w2_hbm.at[j], w2_buf, wsem.at[1]).wait()

    bar = pltpu.get_barrier_semaphore()
    for k in range(1, N):
        pl.semaphore_signal(bar, inc=1, device_id=((me + k) % N,),
                            device_id_type=pl.DeviceIdType.MESH)
    pl.semaphore_wait(bar, N - 1)

    load_w(0)

    rowb = lax.broadcasted_iota(jnp.int32, (CAPE, T), 0)
    row = lax.broadcasted_iota(jnp.int32, (B, T), 0)
    for k in range(1, N):
        Pj[pl.ds(0, CAPE), :] = (
            rowb == slots[...] - base(0, k)).astype(jnp.bfloat16)
        x_send[pl.ds(base(0, k), CAPE), :] = jax.lax.dot_general(
            Pj[pl.ds(0, CAPE), :], x_bf[...], (((1,), (0,)), ((), ())),
            preferred_element_type=jnp.float32).astype(jnp.bfloat16)
        dispatch_rdma(0, k).start()
    Pj[pl.ds(0, CAPE), :] = (
        rowb == slots[...] - base(0, 0)).astype(jnp.bfloat16)
    recv_x[0, 0] = jax.lax.dot_general(
        Pj[pl.ds(0, CAPE), :], x_bf[...], (((1,), (0,)), ((), ())),
        preferred_element_type=jnp.float32).astype(jnp.bfloat16)
    for j in range(1, EL):
        Pj[...] = (row == slots[...] - j * B).astype(jnp.bfloat16)
        r = jax.lax.dot_general(
            Pj[...], x_bf[...], (((1,), (0,)), ((), ())),
            preferred_element_type=jnp.float32).astype(jnp.bfloat16)
        x_send[pl.ds(j * B, B), :] = r
        recv_x[j, 0] = r[:CAPE]
        for k in range(1, N):
            dispatch_rdma(j, k).start()

    colB = lax.broadcasted_iota(jnp.int32, (T, B), 1)
    def unpack_expert(j):
        for k in range(1, N):
            return_rdma(j, k).wait_recv()
        Pc[...] = (slots_t[...] - j * B == colB).astype(jnp.bfloat16)
        contrib = jax.lax.dot_general(
            Pc[...], result_recv[pl.ds(j * B, B), :], (((1,), (0,)), ((), ())),
            preferred_element_type=jnp.float32)
        out[...] = contrib if j == 0 else out[...] + contrib

    for j in range(EL):
        wait_w(j)
        for k in range(1, N):
            dispatch_rdma(j, k).wait_recv()
        xb = recv_x[j].reshape(B, D).astype(jnp.float32)
        h[...] = jnp.maximum(
            jnp.dot(xb, w1_buf[...], preferred_element_type=jnp.float32),
            0.0)
        r = jnp.dot(h[...], w2_buf[...],
                    preferred_element_type=jnp.float32).astype(jnp.bfloat16)
        result_recv[pl.ds(base(j, 0), CAPE), :] = r[:CAPE]
        result_comp[j] = r[CAPE:].reshape(N - 1, CAPE, D)
        for k in range(1, N):
            return_rdma(j, k).start()
        if j + 1 < EL:
            load_w(j + 1)
        if j >= 1:
            unpack_expert(j - 1)
    unpack_expert(EL - 1)

    for j in range(EL):
        for k in range(1, N):
            dispatch_rdma(j, k).wait_send()
            return_rdma(j, k).wait_send()


def kernel(x, assign, W1, W2):
    me = lax.axis_index("i")

    a = assign.astype(jnp.int32)
    oh = (a[:, None] == jnp.arange(16, dtype=jnp.int32)[None, :])
    ranks = jnp.sum(
        jnp.where(oh, jnp.cumsum(oh.astype(jnp.int32), axis=0) - 1, 0),
        axis=1)
    own = a // EL
    jj = a % EL
    kk = (own - me) % N
    slots = (jj * N + kk) * CAPE + ranks
    slots = jnp.where(ranks < CAPE, slots, S)

    out = pl.pallas_call(
        _body,
        out_shape=jax.ShapeDtypeStruct((T, D), jnp.float32),
        in_specs=[
            pl.BlockSpec(memory_space=pltpu.VMEM),
            pl.BlockSpec(memory_space=pltpu.VMEM),
            pl.BlockSpec(memory_space=pltpu.VMEM),
            pl.BlockSpec(memory_space=pl.ANY),
            pl.BlockSpec(memory_space=pl.ANY),
        ],
        out_specs=pl.BlockSpec(memory_space=pltpu.VMEM),
        scratch_shapes=[
            pltpu.VMEM((B, T), jnp.bfloat16),
            pltpu.VMEM((T, B), jnp.bfloat16),
            pltpu.VMEM((S, D), jnp.bfloat16),
            pltpu.VMEM((EL, N, CAPE, D), jnp.bfloat16),
            pltpu.VMEM((EL, N - 1, CAPE, D), jnp.bfloat16),
            pltpu.VMEM((S, D), jnp.bfloat16),
            pltpu.VMEM((D, F), jnp.float32),
            pltpu.VMEM((F, D), jnp.float32),
            pltpu.VMEM((B, F), jnp.float32),
            pltpu.SemaphoreType.DMA((EL, N)),
            pltpu.SemaphoreType.DMA((EL, N)),
            pltpu.SemaphoreType.DMA((EL, N)),
            pltpu.SemaphoreType.DMA((EL, N)),
            pltpu.SemaphoreType.DMA((2,)),
        ],
        compiler_params=pltpu.CompilerParams(
            collective_id=0, vmem_limit_bytes=63 * 1024 * 1024),
    )(x.astype(jnp.bfloat16), slots.reshape(1, T), slots.reshape(T, 1),
      W1, W2)

    return out
